# baseline (speedup 1.0000x reference)
import functools

import jax
import jax.numpy as jnp
from jax import lax
from jax.experimental import pallas as pl
from jax.experimental.pallas import tpu as pltpu
from jax.experimental.pallas import tpu_sc as plsc

_N = 10000
_D = 128
_E = 320000
_NC = 2
_NS = 16
_NW = _NC * _NS
_CHUNK = 128
_CPW = 80
_EPAD = _NW * _CPW * _CHUNK
_NPAD = 10240
_RPT = _NPAD // _NS
_DEGW = 128

_vector_mesh = plsc.VectorSubcoreMesh(core_axis_name="c", subcore_axis_name="s")


@functools.partial(
    pl.kernel,
    out_type=jax.ShapeDtypeStruct((_NC, _NPAD, _DEGW), jnp.float32),
    mesh=_vector_mesh,
    scratch_types=[
        pltpu.VMEM((_CPW, _CHUNK), jnp.int32),
        pltpu.VMEM((_CHUNK, _DEGW), jnp.float32),
        pltpu.VMEM_SHARED((_NPAD, _DEGW), jnp.float32),
    ],
)
def _sc_degree(dst_hbm, ones_hbm, zeros_hbm, out_hbm, dst_c, ones_v, acc_sh):
    cid = lax.axis_index("c")
    sid = lax.axis_index("s")
    wid = sid * _NC + cid
    row0 = sid * _RPT
    pltpu.sync_copy(dst_hbm.at[wid], dst_c)
    for b in range(_RPT // _CHUNK):
        pltpu.sync_copy(zeros_hbm, acc_sh.at[pl.ds(row0 + b * _CHUNK, _CHUNK)])
    pltpu.sync_copy(ones_hbm, ones_v)
    plsc.subcore_barrier()

    @pl.loop(0, _CPW)
    def _(j):
        pltpu.sync_copy(ones_v, acc_sh.at[dst_c.at[j]], add=True)

    plsc.subcore_barrier()
    pltpu.sync_copy(acc_sh.at[pl.ds(row0, _RPT)],
                    out_hbm.at[cid, pl.ds(row0, _RPT)])


@functools.partial(
    pl.kernel,
    out_type=jax.ShapeDtypeStruct((_NC, _NPAD, _D), jnp.float32),
    mesh=_vector_mesh,
    scratch_types=[
        pltpu.VMEM((_CPW, _CHUNK), jnp.int32),
        pltpu.VMEM((_CPW, _CHUNK), jnp.int32),
        pltpu.VMEM((_CHUNK, _D), jnp.float32),
        pltpu.VMEM_SHARED((_NPAD, _D), jnp.float32),
        pltpu.SemaphoreType.DMA,
    ],
)
def _sc_scatter(src_hbm, dst_hbm, g_hbm, zeros_hbm, out_hbm,
                src_c, dst_c, rows_v, acc_sh, sem):
    cid = lax.axis_index("c")
    sid = lax.axis_index("s")
    wid = sid * _NC + cid
    row0 = sid * _RPT
    pltpu.sync_copy(src_hbm.at[wid], src_c)
    pltpu.sync_copy(dst_hbm.at[wid], dst_c)
    for b in range(_RPT // _CHUNK):
        pltpu.sync_copy(zeros_hbm, acc_sh.at[pl.ds(row0 + b * _CHUNK, _CHUNK)])
    plsc.subcore_barrier()

    @pl.loop(0, _CPW)
    def _(j):
        pltpu.async_copy(g_hbm.at[src_c.at[j]], rows_v, sem).wait()
        pltpu.sync_copy(rows_v, acc_sh.at[dst_c.at[j]], add=True)

    plsc.subcore_barrier()
    pltpu.sync_copy(acc_sh.at[pl.ds(row0, _RPT)],
                    out_hbm.at[cid, pl.ds(row0, _RPT)])


_RB = 1000


def _dinv_from(dp):
    deg = dp[0, :, 0:1] + dp[1, :, 0:1] + 1.0
    return lax.rsqrt(deg)


def _mm1_body(dp_ref, x_ref, w_ref, o_ref):
    dinv = _dinv_from(dp_ref[...])
    h = jnp.dot(x_ref[...], w_ref[...], preferred_element_type=jnp.float32)
    o_ref[...] = h * dinv


def _mid_body(dp_ref, sp_ref, g1_ref, b1_ref, w2_ref, o_ref):
    dinv = _dinv_from(dp_ref[...])
    s = sp_ref[0] + sp_ref[1]
    a = jnp.maximum(dinv * (s + g1_ref[...]) + b1_ref[...], 0.0)
    o_ref[...] = jnp.dot(a, w2_ref[...],
                         preferred_element_type=jnp.float32) * dinv


def _fin_body(dp_ref, sp_ref, g2_ref, b2_ref, o_ref):
    dinv = _dinv_from(dp_ref[...])
    o_ref[...] = dinv * (sp_ref[0] + sp_ref[1] + g2_ref[...]) + b2_ref[...]


_dp_spec = pl.BlockSpec((2, _RB, _DEGW), lambda i: (0, i, 0))
_sp_spec = pl.BlockSpec((2, _RB, _D), lambda i: (0, i, 0))
_row_spec = pl.BlockSpec((_RB, _D), lambda i: (i, 0))
_w_spec = pl.BlockSpec((_D, _D), lambda i: (0, 0))
_b_spec = pl.BlockSpec((1, _D), lambda i: (0, 0))
_out_shape = jax.ShapeDtypeStruct((_N, _D), jnp.float32)

_mm1 = pl.pallas_call(
    _mm1_body, grid=(_N // _RB,),
    in_specs=[_dp_spec, _row_spec, _w_spec],
    out_specs=_row_spec, out_shape=_out_shape)

_mid = pl.pallas_call(
    _mid_body, grid=(_N // _RB,),
    in_specs=[_dp_spec, _sp_spec, _row_spec, _b_spec, _w_spec],
    out_specs=_row_spec, out_shape=_out_shape)

_fin = pl.pallas_call(
    _fin_body, grid=(_N // _RB,),
    in_specs=[_dp_spec, _sp_spec, _row_spec, _b_spec],
    out_specs=_row_spec, out_shape=_out_shape)


def kernel(x, edge_index, W1, b1, W2, b2):
    src = edge_index[0].astype(jnp.int32)
    dst = edge_index[1].astype(jnp.int32)
    pad = _EPAD - _E
    ar = jnp.arange(pad, dtype=jnp.int32)
    src_p = jnp.concatenate([src, ar % _N]).reshape(_NW, _CPW, _CHUNK)
    dst_p = jnp.concatenate([dst, _N + (ar % (_NPAD - _N))]
                            ).reshape(_NW, _CPW, _CHUNK)
    ones16 = jnp.ones((_CHUNK, _DEGW), jnp.float32)
    zeros16 = jnp.zeros((_CHUNK, _DEGW), jnp.float32)
    zerosD = jnp.zeros((_CHUNK, _D), jnp.float32)
    b1r = b1.reshape(1, _D)
    b2r = b2.reshape(1, _D)

    dp = _sc_degree(dst_p, ones16, zeros16)
    g1 = _mm1(dp, x, W1)
    s1 = _sc_scatter(src_p, dst_p, g1, zerosD)
    g2 = _mid(dp, s1, g1, b1r, W2)
    s2 = _sc_scatter(src_p, dst_p, g2, zerosD)
    out = _fin(dp, s2, g2, b2r)
    return out


# speedup vs baseline: 1.3306x; 1.3306x over previous
import functools

import jax
import jax.numpy as jnp
from jax import lax
from jax.experimental import pallas as pl
from jax.experimental.pallas import tpu as pltpu
from jax.experimental.pallas import tpu_sc as plsc

_N = 10000
_D = 128
_E = 320000
_NC = 2
_NS = 16
_NW = _NC * _NS
_CHUNK = 128
_CPW = 80
_HCPW = _CPW // 2
_EPAD = _NW * _CPW * _CHUNK
_NPAD = 10240
_RPT = _NPAD // _NS
_DEGW = 128

_vector_mesh = plsc.VectorSubcoreMesh(core_axis_name="c", subcore_axis_name="s")


@functools.partial(
    pl.kernel,
    out_type=jax.ShapeDtypeStruct((_NC, _NPAD, _DEGW), jnp.float32),
    mesh=_vector_mesh,
    scratch_types=[
        pltpu.VMEM((_CPW, _CHUNK), jnp.int32),
        pltpu.VMEM((_CHUNK, _DEGW), jnp.float32),
        pltpu.VMEM_SHARED((_NPAD, _DEGW), jnp.float32),
    ],
)
def _sc_degree(dst_hbm, ones_hbm, zeros_hbm, out_hbm, dst_c, ones_v, acc_sh):
    cid = lax.axis_index("c")
    sid = lax.axis_index("s")
    wid = sid * _NC + cid
    row0 = sid * _RPT
    pltpu.sync_copy(dst_hbm.at[wid], dst_c)
    for b in range(_RPT // _CHUNK):
        pltpu.sync_copy(zeros_hbm, acc_sh.at[pl.ds(row0 + b * _CHUNK, _CHUNK)])
    pltpu.sync_copy(ones_hbm, ones_v)
    plsc.subcore_barrier()

    @pl.loop(0, _CPW)
    def _(j):
        pltpu.sync_copy(ones_v, acc_sh.at[dst_c.at[j]], add=True)

    plsc.subcore_barrier()
    pltpu.sync_copy(acc_sh.at[pl.ds(row0, _RPT)],
                    out_hbm.at[cid, pl.ds(row0, _RPT)])


@functools.partial(
    pl.kernel,
    out_type=jax.ShapeDtypeStruct((_NC, _NPAD, _D), jnp.float32),
    mesh=_vector_mesh,
    scratch_types=[
        pltpu.VMEM((_HCPW, _CHUNK), jnp.int32),
        pltpu.VMEM((_HCPW, _CHUNK), jnp.int32),
        pltpu.VMEM((_CHUNK, _D), jnp.float32),
        pltpu.VMEM((_CHUNK, _D), jnp.float32),
        pltpu.VMEM_SHARED((_NPAD, _D), jnp.float32),
        pltpu.SemaphoreType.DMA,
        pltpu.SemaphoreType.DMA,
    ],
)
def _sc_scatter(src_hbm, dst_hbm, g_hbm, zeros_hbm, out_hbm,
                src_c, dst_c, rows_a, rows_b, acc_sh, sem_a, sem_b):
    cid = lax.axis_index("c")
    sid = lax.axis_index("s")
    wid = sid * _NC + cid
    row0 = sid * _RPT
    for b in range(_RPT // _CHUNK):
        pltpu.sync_copy(zeros_hbm, acc_sh.at[pl.ds(row0 + b * _CHUNK, _CHUNK)])
    plsc.subcore_barrier()

    for h in range(2):
        base = h * _HCPW
        pltpu.sync_copy(src_hbm.at[wid, pl.ds(base, _HCPW)], src_c)
        pltpu.sync_copy(dst_hbm.at[wid, pl.ds(base, _HCPW)], dst_c)
        pltpu.async_copy(g_hbm.at[src_c.at[0]], rows_a, sem_a)

        @pl.loop(0, _HCPW // 2 - 1)
        def _(p):
            j = 2 * p
            pltpu.async_copy(g_hbm.at[src_c.at[j + 1]], rows_b, sem_b)
            pltpu.make_async_copy(g_hbm.at[src_c.at[j]], rows_a, sem_a).wait()
            pltpu.sync_copy(rows_a, acc_sh.at[dst_c.at[j]], add=True)
            pltpu.async_copy(g_hbm.at[src_c.at[j + 2]], rows_a, sem_a)
            pltpu.make_async_copy(g_hbm.at[src_c.at[j + 1]], rows_b,
                                  sem_b).wait()
            pltpu.sync_copy(rows_b, acc_sh.at[dst_c.at[j + 1]], add=True)

        pltpu.async_copy(g_hbm.at[src_c.at[_HCPW - 1]], rows_b, sem_b)
        pltpu.make_async_copy(g_hbm.at[src_c.at[_HCPW - 2]], rows_a,
                              sem_a).wait()
        pltpu.sync_copy(rows_a, acc_sh.at[dst_c.at[_HCPW - 2]], add=True)
        pltpu.make_async_copy(g_hbm.at[src_c.at[_HCPW - 1]], rows_b,
                              sem_b).wait()
        pltpu.sync_copy(rows_b, acc_sh.at[dst_c.at[_HCPW - 1]], add=True)

    plsc.subcore_barrier()
    pltpu.sync_copy(acc_sh.at[pl.ds(row0, _RPT)],
                    out_hbm.at[cid, pl.ds(row0, _RPT)])


_RB = 1000


def _dinv_from(dp):
    deg = dp[0, :, 0:1] + dp[1, :, 0:1] + 1.0
    return lax.rsqrt(deg)


def _mm1_body(dp_ref, x_ref, w_ref, o_ref):
    dinv = _dinv_from(dp_ref[...])
    h = jnp.dot(x_ref[...], w_ref[...], preferred_element_type=jnp.float32)
    o_ref[...] = h * dinv


def _mid_body(dp_ref, sp_ref, g1_ref, b1_ref, w2_ref, o_ref):
    dinv = _dinv_from(dp_ref[...])
    s = sp_ref[0] + sp_ref[1]
    a = jnp.maximum(dinv * (s + g1_ref[...]) + b1_ref[...], 0.0)
    o_ref[...] = jnp.dot(a, w2_ref[...],
                         preferred_element_type=jnp.float32) * dinv


def _fin_body(dp_ref, sp_ref, g2_ref, b2_ref, o_ref):
    dinv = _dinv_from(dp_ref[...])
    o_ref[...] = dinv * (sp_ref[0] + sp_ref[1] + g2_ref[...]) + b2_ref[...]


_dp_spec = pl.BlockSpec((2, _RB, _DEGW), lambda i: (0, i, 0))
_sp_spec = pl.BlockSpec((2, _RB, _D), lambda i: (0, i, 0))
_row_spec = pl.BlockSpec((_RB, _D), lambda i: (i, 0))
_w_spec = pl.BlockSpec((_D, _D), lambda i: (0, 0))
_b_spec = pl.BlockSpec((1, _D), lambda i: (0, 0))
_out_shape = jax.ShapeDtypeStruct((_N, _D), jnp.float32)

_mm1 = pl.pallas_call(
    _mm1_body, grid=(_N // _RB,),
    in_specs=[_dp_spec, _row_spec, _w_spec],
    out_specs=_row_spec, out_shape=_out_shape)

_mid = pl.pallas_call(
    _mid_body, grid=(_N // _RB,),
    in_specs=[_dp_spec, _sp_spec, _row_spec, _b_spec, _w_spec],
    out_specs=_row_spec, out_shape=_out_shape)

_fin = pl.pallas_call(
    _fin_body, grid=(_N // _RB,),
    in_specs=[_dp_spec, _sp_spec, _row_spec, _b_spec],
    out_specs=_row_spec, out_shape=_out_shape)


def kernel(x, edge_index, W1, b1, W2, b2):
    src = edge_index[0].astype(jnp.int32)
    dst = edge_index[1].astype(jnp.int32)
    pad = _EPAD - _E
    ar = jnp.arange(pad, dtype=jnp.int32)
    src_p = jnp.concatenate([src, ar % _N]).reshape(_NW, _CPW, _CHUNK)
    dst_p = jnp.concatenate([dst, _N + (ar % (_NPAD - _N))]
                            ).reshape(_NW, _CPW, _CHUNK)
    ones16 = jnp.ones((_CHUNK, _DEGW), jnp.float32)
    zeros16 = jnp.zeros((_CHUNK, _DEGW), jnp.float32)
    zerosD = jnp.zeros((_CHUNK, _D), jnp.float32)
    b1r = b1.reshape(1, _D)
    b2r = b2.reshape(1, _D)

    dp = _sc_degree(dst_p, ones16, zeros16)
    g1 = _mm1(dp, x, W1)
    s1 = _sc_scatter(src_p, dst_p, g1, zerosD)
    g2 = _mid(dp, s1, g1, b1r, W2)
    s2 = _sc_scatter(src_p, dst_p, g2, zerosD)
    out = _fin(dp, s2, g2, b2r)
    return out
